# Initial kernel scaffold; baseline (speedup 1.0000x reference)
import jax
import jax.numpy as jnp
from jax.experimental import pallas as pl


def kernel(fragments, zbuf, ptclds):
    raise NotImplementedError("write your pallas kernel here")


# speedup vs baseline: 14.3030x; 14.3030x over previous
import functools

import jax
import jax.numpy as jnp
from jax import lax
from jax.experimental import pallas as pl
from jax.experimental.pallas import tpu as pltpu
from jax.experimental.pallas import tpu_sc as plsc

N, K, H, W = 4, 8, 384, 384
C = 16
NPTS = 100000
NC, NS, L = 2, 16, 16
NWORKERS = NC * NS
PIX = H * W
PER_W = PIX // NWORKERS
P = 512
CHUNKS = PER_W // P

_mesh = plsc.VectorSubcoreMesh(core_axis_name="c", subcore_axis_name="s")


@functools.partial(
    pl.kernel,
    out_type=jax.ShapeDtypeStruct((N, C, PIX), jnp.float32),
    mesh=_mesh,
    compiler_params=pltpu.CompilerParams(
        needs_layout_passes=False, use_tc_tiling_on_sc=False),
    scratch_types=[
        [pltpu.VMEM((P,), jnp.int32) for _ in range(K)],
        pltpu.VMEM((K, P), jnp.float32),
        pltpu.VMEM((K, P, C), jnp.float32),
        pltpu.VMEM((C, P), jnp.float32),
        pltpu.SemaphoreType.DMA,
    ],
)
def _compose(table_hbm, frag_hbm, zbuf_hbm, out_hbm, idx_vs, z_v, rows_v,
             out_v, sem):
    wid = lax.axis_index("s") * NC + lax.axis_index("c")

    def chunk_body(t, carry):
        n = t // CHUNKS
        j = t % CHUNKS
        base = wid * PER_W + j * P
        for k in range(K):
            pltpu.sync_copy(frag_hbm.at[n, k, pl.ds(base, P)], idx_vs[k])
        pltpu.sync_copy(zbuf_hbm.at[n, :, pl.ds(base, P)], z_v)
        cps = [pltpu.async_copy(table_hbm.at[idx_vs[k]], rows_v.at[k], sem)
               for k in range(K)]
        for cp in cps:
            cp.wait()

        def group_body(g, carry2):
            p0 = g * L
            lanes = lax.iota(jnp.int32, L)
            imps = []
            for k in range(K):
                zk = z_v[k, pl.ds(p0, L)]
                zp = jnp.where(zk < 0.0, jnp.float32(-0.0001), zk)
                imps.append(1.0 / (zp + 1e-6))
            m = imps[0]
            for k in range(1, K):
                m = jnp.maximum(m, imps[k])
            es = [jnp.exp(i - m) for i in imps]
            s = es[0]
            for k in range(1, K):
                s = s + es[k]
            inv = 1.0 / s
            ws = [e * inv for e in es]
            rows = lanes + p0
            for c in range(C):
                ci = jnp.full((L,), c, jnp.int32)
                acc = None
                for k in range(K):
                    ki = jnp.full((L,), k, jnp.int32)
                    gk = plsc.load_gather(rows_v, [ki, rows, ci])
                    acc = ws[k] * gk if acc is None else acc + ws[k] * gk
                out_v[c, pl.ds(p0, L)] = acc
            return carry2

        lax.fori_loop(0, P // L, group_body, 0)
        pltpu.sync_copy(out_v, out_hbm.at[n, :, pl.ds(base, P)])
        return carry

    lax.fori_loop(0, N * CHUNKS, chunk_body, 0)


def kernel(fragments, zbuf, ptclds):
    table = jnp.transpose(ptclds)
    frag = fragments.reshape(N, K, PIX)
    zb = zbuf.reshape(N, K, PIX)
    out = _compose(table, frag, zb)
    return out.reshape(N, C, H, W)
